# Initial kernel scaffold; baseline (speedup 1.0000x reference)
import jax
import jax.numpy as jnp
from jax.experimental import pallas as pl


def kernel(label, depth, batch, emb, W_iou, U_iou, b_iou, U_f_w, U_f_b, out_w, out_b):
    raise NotImplementedError("write your pallas kernel here")


# speedup vs baseline: 14.3600x; 14.3600x over previous
import functools

import jax
import jax.numpy as jnp
from jax import lax
from jax.experimental import pallas as pl
from jax.experimental.pallas import tpu as pltpu
from jax.experimental.pallas import tpu_sc as plsc

H = 256


@functools.lru_cache(maxsize=None)
def _make_sc_gather(V, N, CH):
    info = plsc.get_sparse_core_info()
    NC, NS = info.num_cores, info.num_subcores
    NW = NC * NS
    per_w = N // NW
    n_ch = per_w // CH
    assert per_w % CH == 0 and N % NW == 0 and CH <= 128
    mesh = plsc.VectorSubcoreMesh(core_axis_name="c", subcore_axis_name="s")

    @functools.partial(
        pl.kernel,
        mesh=mesh,
        out_type=jax.ShapeDtypeStruct((N, H), jnp.float32),
        scratch_types=[
            pltpu.VMEM((n_ch, CH), jnp.int32),
            pltpu.VMEM((2, CH, H), jnp.float32),
            pltpu.SemaphoreType.DMA,
            pltpu.SemaphoreType.DMA,
            pltpu.SemaphoreType.DMA,
            pltpu.SemaphoreType.DMA,
        ],
    )
    def gather_k(table_hbm, idx_hbm, out_hbm, idx_v, rows_v, g0, g1, o0, o1):
        wid = lax.axis_index("s") * NC + lax.axis_index("c")
        base = wid * per_w
        gsems = (g0, g1)
        osems = (o0, o1)

        pltpu.sync_copy(idx_hbm.at[wid], idx_v)
        gathers = [None, None]
        wbs = [None, None]
        gathers[0] = pltpu.async_copy(table_hbm.at[idx_v.at[0]], rows_v.at[0], g0)
        for k in range(n_ch):
            b = k % 2
            if k + 1 < n_ch:
                nb = 1 - b
                if wbs[nb] is not None:
                    wbs[nb].wait()
                gathers[nb] = pltpu.async_copy(
                    table_hbm.at[idx_v.at[k + 1]], rows_v.at[nb], gsems[nb]
                )
            gathers[b].wait()
            wbs[b] = pltpu.async_copy(
                rows_v.at[b], out_hbm.at[pl.ds(base + k * CH, CH)], osems[b]
            )
        for w in wbs:
            if w is not None:
                w.wait()

    return gather_k


def _leaf_body(x_ref, w_ref, b_ref, h_ref, c_ref):
    iou = jnp.dot(x_ref[...], w_ref[...], preferred_element_type=jnp.float32)
    iou = iou + b_ref[...]
    i, o, u = iou[:, :H], iou[:, H:2 * H], iou[:, 2 * H:]
    c = jax.nn.sigmoid(i) * jnp.tanh(u)
    h_ref[...] = jax.nn.sigmoid(o) * jnp.tanh(c)
    c_ref[...] = c


def _leaf_call(x, WiouT, b_iou):
    N = x.shape[0]
    BM = 512
    grid = (N // BM,)
    return pl.pallas_call(
        _leaf_body,
        grid=grid,
        in_specs=[
            pl.BlockSpec((BM, H), lambda i: (i, 0)),
            pl.BlockSpec((H, 3 * H), lambda i: (0, 0)),
            pl.BlockSpec((1, 3 * H), lambda i: (0, 0)),
        ],
        out_specs=[
            pl.BlockSpec((BM, H), lambda i: (i, 0)),
            pl.BlockSpec((BM, H), lambda i: (i, 0)),
        ],
        out_shape=[jax.ShapeDtypeStruct((N, H), jnp.float32)] * 2,
    )(x, WiouT, b_iou)


def _level_body(h_ref, c_ref, wf_ref, bf_ref, wi_ref, bi_ref, hn_ref, cn_ref):
    hcat = h_ref[...]
    f = jax.nn.sigmoid(
        jnp.dot(hcat, wf_ref[...], preferred_element_type=jnp.float32) + bf_ref[...]
    )
    fc = f * c_ref[...]
    ct = fc[:, :H] + fc[:, H:]
    iou = jnp.dot(hcat, wi_ref[...], preferred_element_type=jnp.float32) + bi_ref[...]
    i, o, u = iou[:, :H], iou[:, H:2 * H], iou[:, 2 * H:]
    cn = jax.nn.sigmoid(i) * jnp.tanh(u) + ct
    hn_ref[...] = jax.nn.sigmoid(o) * jnp.tanh(cn)
    cn_ref[...] = cn


def _level_call(h2, c2, WfT, bf, WiouT, biou):
    m = h2.shape[0]
    BM = min(m, 512)
    grid = (m // BM,)
    return pl.pallas_call(
        _level_body,
        grid=grid,
        in_specs=[
            pl.BlockSpec((BM, 2 * H), lambda i: (i, 0)),
            pl.BlockSpec((BM, 2 * H), lambda i: (i, 0)),
            pl.BlockSpec((2 * H, 2 * H), lambda i: (0, 0)),
            pl.BlockSpec((1, 2 * H), lambda i: (0, 0)),
            pl.BlockSpec((2 * H, 3 * H), lambda i: (0, 0)),
            pl.BlockSpec((1, 3 * H), lambda i: (0, 0)),
        ],
        out_specs=[
            pl.BlockSpec((BM, H), lambda i: (i, 0)),
            pl.BlockSpec((BM, H), lambda i: (i, 0)),
        ],
        out_shape=[jax.ShapeDtypeStruct((m, H), jnp.float32)] * 2,
    )(h2, c2, WfT, bf, WiouT, biou)


def _root_body(h_ref, c_ref, wf_ref, bf_ref, wi_ref, bi_ref, wo_ref, bo_ref, out_ref):
    hcat = h_ref[...]
    f = jax.nn.sigmoid(
        jnp.dot(hcat, wf_ref[...], preferred_element_type=jnp.float32) + bf_ref[...]
    )
    fc = f * c_ref[...]
    ct = fc[:, :H] + fc[:, H:]
    iou = jnp.dot(hcat, wi_ref[...], preferred_element_type=jnp.float32) + bi_ref[...]
    i, o, u = iou[:, :H], iou[:, H:2 * H], iou[:, 2 * H:]
    cn = jax.nn.sigmoid(i) * jnp.tanh(u) + ct
    hn = jax.nn.sigmoid(o) * jnp.tanh(cn)
    logits = jnp.dot(hn, wo_ref[...], preferred_element_type=jnp.float32) + bo_ref[...]
    mx = jnp.max(logits, axis=-1, keepdims=True)
    e = jnp.exp(logits - mx)
    s = jnp.sum(e, axis=-1, keepdims=True)
    out_ref[...] = logits - mx - jnp.log(s)


def _root_call(h2, c2, WfT, bf, WiouT, biou, WoPad, boPad):
    m = h2.shape[0]
    return pl.pallas_call(
        _root_body,
        grid=(1,),
        in_specs=[
            pl.BlockSpec((m, 2 * H), lambda i: (0, 0)),
            pl.BlockSpec((m, 2 * H), lambda i: (0, 0)),
            pl.BlockSpec((2 * H, 2 * H), lambda i: (0, 0)),
            pl.BlockSpec((1, 2 * H), lambda i: (0, 0)),
            pl.BlockSpec((2 * H, 3 * H), lambda i: (0, 0)),
            pl.BlockSpec((1, 3 * H), lambda i: (0, 0)),
            pl.BlockSpec((H, 128), lambda i: (0, 0)),
            pl.BlockSpec((1, 128), lambda i: (0, 0)),
        ],
        out_specs=pl.BlockSpec((m, 128), lambda i: (0, 0)),
        out_shape=jax.ShapeDtypeStruct((m, 128), jnp.float32),
    )(h2, c2, WfT, bf, WiouT, biou, WoPad, boPad)


def kernel(label, depth, batch, emb, W_iou, U_iou, b_iou, U_f_w, U_f_b, out_w, out_b):
    D = 10
    M = 2 ** (D + 1) - 1
    B = label.shape[0] // M
    NL = B * 2 ** D
    out_size = out_w.shape[0]

    leaf_labels = label.reshape(B, M)[:, M // 2:].reshape(-1).astype(jnp.int32)

    CH = 128
    info = plsc.get_sparse_core_info()
    NW = info.num_cores * info.num_subcores
    idx3 = leaf_labels.reshape(NW, NL // (NW * CH), CH)
    x = _make_sc_gather(emb.shape[0], NL, CH)(emb, idx3)

    WiouLeafT = W_iou.T
    WfT = U_f_w.T
    WiouT = U_iou.T
    bf = U_f_b.reshape(1, 2 * H)
    biou = b_iou.reshape(1, 3 * H)

    h, c = _leaf_call(x, WiouLeafT, biou)

    for l in range(D - 1, 0, -1):
        m = B * 2 ** l
        h2 = h.reshape(m, 2 * H)
        c2 = c.reshape(m, 2 * H)
        h, c = _level_call(h2, c2, WfT, bf, WiouT, biou)

    WoPad = jnp.zeros((H, 128), jnp.float32).at[:, :out_size].set(out_w.T)
    boPad = jnp.full((1, 128), -1e30, jnp.float32).at[0, :out_size].set(out_b)
    ls = _root_call(h.reshape(B, 2 * H), c.reshape(B, 2 * H), WfT, bf, WiouT, biou,
                    WoPad, boPad)
    return ls[:, :out_size]


# speedup vs baseline: 14.3680x; 1.0006x over previous
import functools

import jax
import jax.numpy as jnp
from jax import lax
from jax.experimental import pallas as pl
from jax.experimental.pallas import tpu as pltpu
from jax.experimental.pallas import tpu_sc as plsc

H = 256


@functools.lru_cache(maxsize=None)
def _make_sc_gather(V, N, CH):
    info = plsc.get_sparse_core_info()
    NC, NS = info.num_cores, info.num_subcores
    NW = NC * NS
    per_w = N // NW
    n_ch = per_w // CH
    assert per_w % CH == 0 and N % NW == 0 and CH <= 128
    mesh = plsc.VectorSubcoreMesh(core_axis_name="c", subcore_axis_name="s")

    @functools.partial(
        pl.kernel,
        mesh=mesh,
        out_type=jax.ShapeDtypeStruct((N, H), jnp.float32),
        scratch_types=[
            pltpu.VMEM((n_ch, CH), jnp.int32),
            pltpu.VMEM((2, CH, H), jnp.float32),
            pltpu.SemaphoreType.DMA,
            pltpu.SemaphoreType.DMA,
            pltpu.SemaphoreType.DMA,
            pltpu.SemaphoreType.DMA,
        ],
    )
    def gather_k(table_hbm, idx_hbm, out_hbm, idx_v, rows_v, g0, g1, o0, o1):
        wid = lax.axis_index("s") * NC + lax.axis_index("c")
        base = wid * per_w
        gsems = (g0, g1)
        osems = (o0, o1)

        pltpu.sync_copy(idx_hbm.at[wid], idx_v)
        gathers = [None, None]
        wbs = [None, None]
        gathers[0] = pltpu.async_copy(table_hbm.at[idx_v.at[0]], rows_v.at[0], g0)
        for k in range(n_ch):
            b = k % 2
            if k + 1 < n_ch:
                nb = 1 - b
                if wbs[nb] is not None:
                    wbs[nb].wait()
                gathers[nb] = pltpu.async_copy(
                    table_hbm.at[idx_v.at[k + 1]], rows_v.at[nb], gsems[nb]
                )
            gathers[b].wait()
            wbs[b] = pltpu.async_copy(
                rows_v.at[b], out_hbm.at[pl.ds(base + k * CH, CH)], osems[b]
            )
        for w in wbs:
            if w is not None:
                w.wait()

    return gather_k


def _dot(a, b):
    return jnp.dot(a.astype(jnp.bfloat16), b.astype(jnp.bfloat16),
                   preferred_element_type=jnp.float32)


def _leaf_body(x_ref, w_ref, b_ref, h_ref, c_ref):
    iou = _dot(x_ref[...], w_ref[...])
    iou = iou + b_ref[...]
    i, o, u = iou[:, :H], iou[:, H:2 * H], iou[:, 2 * H:]
    c = jax.nn.sigmoid(i) * jnp.tanh(u)
    h_ref[...] = jax.nn.sigmoid(o) * jnp.tanh(c)
    c_ref[...] = c


def _leaf_call(x, WiouT, b_iou):
    N = x.shape[0]
    BM = 512
    grid = (N // BM,)
    return pl.pallas_call(
        _leaf_body,
        grid=grid,
        in_specs=[
            pl.BlockSpec((BM, H), lambda i: (i, 0)),
            pl.BlockSpec((H, 3 * H), lambda i: (0, 0)),
            pl.BlockSpec((1, 3 * H), lambda i: (0, 0)),
        ],
        out_specs=[
            pl.BlockSpec((BM, H), lambda i: (i, 0)),
            pl.BlockSpec((BM, H), lambda i: (i, 0)),
        ],
        out_shape=[jax.ShapeDtypeStruct((N, H), jnp.float32)] * 2,
    )(x, WiouT, b_iou)


def _level_body(h_ref, c_ref, wf_ref, bf_ref, wi_ref, bi_ref, hn_ref, cn_ref):
    hcat = h_ref[...]
    f = jax.nn.sigmoid(_dot(hcat, wf_ref[...]) + bf_ref[...])
    fc = f * c_ref[...]
    ct = fc[:, :H] + fc[:, H:]
    iou = _dot(hcat, wi_ref[...]) + bi_ref[...]
    i, o, u = iou[:, :H], iou[:, H:2 * H], iou[:, 2 * H:]
    cn = jax.nn.sigmoid(i) * jnp.tanh(u) + ct
    hn_ref[...] = jax.nn.sigmoid(o) * jnp.tanh(cn)
    cn_ref[...] = cn


def _level_call(h2, c2, WfT, bf, WiouT, biou):
    m = h2.shape[0]
    BM = min(m, 512)
    grid = (m // BM,)
    return pl.pallas_call(
        _level_body,
        grid=grid,
        in_specs=[
            pl.BlockSpec((BM, 2 * H), lambda i: (i, 0)),
            pl.BlockSpec((BM, 2 * H), lambda i: (i, 0)),
            pl.BlockSpec((2 * H, 2 * H), lambda i: (0, 0)),
            pl.BlockSpec((1, 2 * H), lambda i: (0, 0)),
            pl.BlockSpec((2 * H, 3 * H), lambda i: (0, 0)),
            pl.BlockSpec((1, 3 * H), lambda i: (0, 0)),
        ],
        out_specs=[
            pl.BlockSpec((BM, H), lambda i: (i, 0)),
            pl.BlockSpec((BM, H), lambda i: (i, 0)),
        ],
        out_shape=[jax.ShapeDtypeStruct((m, H), jnp.float32)] * 2,
    )(h2, c2, WfT, bf, WiouT, biou)


def _root_body(h_ref, c_ref, wf_ref, bf_ref, wi_ref, bi_ref, wo_ref, bo_ref, out_ref):
    hcat = h_ref[...]
    f = jax.nn.sigmoid(_dot(hcat, wf_ref[...]) + bf_ref[...])
    fc = f * c_ref[...]
    ct = fc[:, :H] + fc[:, H:]
    iou = _dot(hcat, wi_ref[...]) + bi_ref[...]
    i, o, u = iou[:, :H], iou[:, H:2 * H], iou[:, 2 * H:]
    cn = jax.nn.sigmoid(i) * jnp.tanh(u) + ct
    hn = jax.nn.sigmoid(o) * jnp.tanh(cn)
    logits = jnp.dot(hn, wo_ref[...], preferred_element_type=jnp.float32) + bo_ref[...]
    mx = jnp.max(logits, axis=-1, keepdims=True)
    e = jnp.exp(logits - mx)
    s = jnp.sum(e, axis=-1, keepdims=True)
    out_ref[...] = logits - mx - jnp.log(s)


def _root_call(h2, c2, WfT, bf, WiouT, biou, WoPad, boPad):
    m = h2.shape[0]
    return pl.pallas_call(
        _root_body,
        grid=(1,),
        in_specs=[
            pl.BlockSpec((m, 2 * H), lambda i: (0, 0)),
            pl.BlockSpec((m, 2 * H), lambda i: (0, 0)),
            pl.BlockSpec((2 * H, 2 * H), lambda i: (0, 0)),
            pl.BlockSpec((1, 2 * H), lambda i: (0, 0)),
            pl.BlockSpec((2 * H, 3 * H), lambda i: (0, 0)),
            pl.BlockSpec((1, 3 * H), lambda i: (0, 0)),
            pl.BlockSpec((H, 128), lambda i: (0, 0)),
            pl.BlockSpec((1, 128), lambda i: (0, 0)),
        ],
        out_specs=pl.BlockSpec((m, 128), lambda i: (0, 0)),
        out_shape=jax.ShapeDtypeStruct((m, 128), jnp.float32),
    )(h2, c2, WfT, bf, WiouT, biou, WoPad, boPad)


def kernel(label, depth, batch, emb, W_iou, U_iou, b_iou, U_f_w, U_f_b, out_w, out_b):
    D = 10
    M = 2 ** (D + 1) - 1
    B = label.shape[0] // M
    NL = B * 2 ** D
    out_size = out_w.shape[0]

    leaf_labels = label.reshape(B, M)[:, M // 2:].reshape(-1).astype(jnp.int32)

    CH = 128
    info = plsc.get_sparse_core_info()
    NW = info.num_cores * info.num_subcores
    idx3 = leaf_labels.reshape(NW, NL // (NW * CH), CH)
    x = _make_sc_gather(emb.shape[0], NL, CH)(emb, idx3)

    WiouLeafT = W_iou.T
    WfT = U_f_w.T
    WiouT = U_iou.T
    bf = U_f_b.reshape(1, 2 * H)
    biou = b_iou.reshape(1, 3 * H)

    h, c = _leaf_call(x, WiouLeafT, biou)

    for l in range(D - 1, 0, -1):
        m = B * 2 ** l
        h2 = h.reshape(m, 2 * H)
        c2 = c.reshape(m, 2 * H)
        h, c = _level_call(h2, c2, WfT, bf, WiouT, biou)

    WoPad = jnp.zeros((H, 128), jnp.float32).at[:, :out_size].set(out_w.T)
    boPad = jnp.full((1, 128), -1e30, jnp.float32).at[0, :out_size].set(out_b)
    ls = _root_call(h.reshape(B, 2 * H), c.reshape(B, 2 * H), WfT, bf, WiouT, biou,
                    WoPad, boPad)
    return ls[:, :out_size]


# speedup vs baseline: 40.1148x; 2.7920x over previous
import functools

import jax
import jax.numpy as jnp
from jax import lax
from jax.experimental import pallas as pl
from jax.experimental.pallas import tpu as pltpu
from jax.experimental.pallas import tpu_sc as plsc

H = 256


@functools.lru_cache(maxsize=None)
def _make_sc_gather(V, N, CH):
    info = plsc.get_sparse_core_info()
    NC, NS = info.num_cores, info.num_subcores
    NW = NC * NS
    per_w = N // NW
    n_ch = per_w // CH
    assert per_w % CH == 0 and N % NW == 0 and CH <= 128
    mesh = plsc.VectorSubcoreMesh(core_axis_name="c", subcore_axis_name="s")

    @functools.partial(
        pl.kernel,
        mesh=mesh,
        out_type=jax.ShapeDtypeStruct((N, H), jnp.float32),
        scratch_types=[
            pltpu.VMEM((n_ch, CH), jnp.int32),
            pltpu.VMEM((2, CH, H), jnp.float32),
            pltpu.SemaphoreType.DMA,
            pltpu.SemaphoreType.DMA,
            pltpu.SemaphoreType.DMA,
            pltpu.SemaphoreType.DMA,
        ],
    )
    def gather_k(table_hbm, idx_hbm, out_hbm, idx_v, rows_v, g0, g1, o0, o1):
        wid = lax.axis_index("s") * NC + lax.axis_index("c")
        base = wid * per_w
        gsems = (g0, g1)
        osems = (o0, o1)

        pltpu.sync_copy(idx_hbm.at[wid], idx_v)
        gathers = [None, None]
        wbs = [None, None]
        gathers[0] = pltpu.async_copy(table_hbm.at[idx_v.at[0]], rows_v.at[0], g0)
        for k in range(n_ch):
            b = k % 2
            if k + 1 < n_ch:
                nb = 1 - b
                if wbs[nb] is not None:
                    wbs[nb].wait()
                gathers[nb] = pltpu.async_copy(
                    table_hbm.at[idx_v.at[k + 1]], rows_v.at[nb], gsems[nb]
                )
            gathers[b].wait()
            wbs[b] = pltpu.async_copy(
                rows_v.at[b], out_hbm.at[pl.ds(base + k * CH, CH)], osems[b]
            )
        for w in wbs:
            if w is not None:
                w.wait()

    return gather_k


def _dot(a, b):
    return jnp.dot(a.astype(jnp.bfloat16), b.astype(jnp.bfloat16),
                   preferred_element_type=jnp.float32)


def _lstm_cell(iou, ct):
    i, o, u = iou[:, :H], iou[:, H:2 * H], iou[:, 2 * H:]
    c = jax.nn.sigmoid(i) * jnp.tanh(u) + ct
    h = jax.nn.sigmoid(o) * jnp.tanh(c)
    return h, c


def _levels(h, c, w_ref, b_ref, n_lev):
    for _ in range(n_lev):
        k = h.shape[0] // 2
        h2 = h.reshape(k, 2 * H)
        c2 = c.reshape(k, 2 * H)
        g = _dot(h2, w_ref[...]) + b_ref[...]
        f = jax.nn.sigmoid(g[:, :2 * H])
        fc = f * c2
        ct = fc[:, :H] + fc[:, H:]
        h, c = _lstm_cell(g[:, 2 * H:], ct)
    return h, c


def _blockA_body(x_ref, wl_ref, w_ref, b_ref, h_ref, c_ref):
    iou = _dot(x_ref[...], wl_ref[...]) + b_ref[...][:, 2 * H:]
    h, c = _lstm_cell(iou, 0.0)
    h, c = _levels(h, c, w_ref, b_ref, 4)
    h_ref[...] = h
    c_ref[...] = c


def _tailB_body(hin_ref, cin_ref, w_ref, b_ref, wo_ref, bo_ref, out_ref):
    h, c = _levels(hin_ref[...], cin_ref[...], w_ref, b_ref, 6)
    logits = jnp.dot(h, wo_ref[...], preferred_element_type=jnp.float32) + bo_ref[...]
    mx = jnp.max(logits, axis=-1, keepdims=True)
    e = jnp.exp(logits - mx)
    out_ref[...] = logits - mx - jnp.log(jnp.sum(e, axis=-1, keepdims=True))


def kernel(label, depth, batch, emb, W_iou, U_iou, b_iou, U_f_w, U_f_b, out_w, out_b):
    D = 10
    M = 2 ** (D + 1) - 1
    B = label.shape[0] // M
    NL = B * 2 ** D
    out_size = out_w.shape[0]

    leaf_labels = label.reshape(B, M)[:, M // 2:].reshape(-1).astype(jnp.int32)

    CH = 128
    info = plsc.get_sparse_core_info()
    NW = info.num_cores * info.num_subcores
    idx3 = leaf_labels.reshape(NW, NL // (NW * CH), CH)
    x = _make_sc_gather(emb.shape[0], NL, CH)(emb, idx3)

    WlT = W_iou.T
    Wall = jnp.concatenate([U_f_w.T, U_iou.T], axis=1)
    ball = jnp.concatenate([U_f_b.reshape(1, 2 * H), b_iou.reshape(1, 3 * H)],
                           axis=1)

    T = 4
    LB = T * 2 ** D
    OB = T * 2 ** 6
    h6, c6 = pl.pallas_call(
        _blockA_body,
        grid=(B // T,),
        in_specs=[
            pl.BlockSpec((LB, H), lambda i: (i, 0)),
            pl.BlockSpec((H, 3 * H), lambda i: (0, 0)),
            pl.BlockSpec((2 * H, 5 * H), lambda i: (0, 0)),
            pl.BlockSpec((1, 5 * H), lambda i: (0, 0)),
        ],
        out_specs=[
            pl.BlockSpec((OB, H), lambda i: (i, 0)),
            pl.BlockSpec((OB, H), lambda i: (i, 0)),
        ],
        out_shape=[jax.ShapeDtypeStruct((B * 2 ** 6, H), jnp.float32)] * 2,
    )(x, WlT, Wall, ball)

    WoPad = jnp.zeros((H, 128), jnp.float32).at[:, :out_size].set(out_w.T)
    boPad = jnp.full((1, 128), -1e30, jnp.float32).at[0, :out_size].set(out_b)
    ls = pl.pallas_call(
        _tailB_body,
        grid=(1,),
        in_specs=[
            pl.BlockSpec((B * 2 ** 6, H), lambda i: (0, 0)),
            pl.BlockSpec((B * 2 ** 6, H), lambda i: (0, 0)),
            pl.BlockSpec((2 * H, 5 * H), lambda i: (0, 0)),
            pl.BlockSpec((1, 5 * H), lambda i: (0, 0)),
            pl.BlockSpec((H, 128), lambda i: (0, 0)),
            pl.BlockSpec((1, 128), lambda i: (0, 0)),
        ],
        out_specs=pl.BlockSpec((B, 128), lambda i: (0, 0)),
        out_shape=jax.ShapeDtypeStruct((B, 128), jnp.float32),
    )(h6, c6, Wall, ball, WoPad, boPad)
    return ls[:, :out_size]


# speedup vs baseline: 42.2363x; 1.0529x over previous
import functools

import jax
import jax.numpy as jnp
from jax import lax
from jax.experimental import pallas as pl
from jax.experimental.pallas import tpu as pltpu
from jax.experimental.pallas import tpu_sc as plsc

H = 256


@functools.lru_cache(maxsize=None)
def _make_sc_gather(V, N, CH):
    info = plsc.get_sparse_core_info()
    NC, NS = info.num_cores, info.num_subcores
    NW = NC * NS
    per_w = N // NW
    n_ch = per_w // CH
    assert per_w % CH == 0 and N % NW == 0 and CH <= 128
    mesh = plsc.VectorSubcoreMesh(core_axis_name="c", subcore_axis_name="s")

    @functools.partial(
        pl.kernel,
        mesh=mesh,
        out_type=jax.ShapeDtypeStruct((N, H), jnp.float32),
        scratch_types=[
            pltpu.VMEM((n_ch, CH), jnp.int32),
            pltpu.VMEM((2, CH, H), jnp.float32),
            pltpu.SemaphoreType.DMA,
            pltpu.SemaphoreType.DMA,
            pltpu.SemaphoreType.DMA,
            pltpu.SemaphoreType.DMA,
        ],
    )
    def gather_k(table_hbm, idx_hbm, out_hbm, idx_v, rows_v, g0, g1, o0, o1):
        wid = lax.axis_index("s") * NC + lax.axis_index("c")
        base = wid * per_w
        gsems = (g0, g1)
        osems = (o0, o1)

        pltpu.sync_copy(idx_hbm.at[wid], idx_v)
        gathers = [None, None]
        wbs = [None, None]
        gathers[0] = pltpu.async_copy(table_hbm.at[idx_v.at[0]], rows_v.at[0], g0)
        for k in range(n_ch):
            b = k % 2
            if k + 1 < n_ch:
                nb = 1 - b
                if wbs[nb] is not None:
                    wbs[nb].wait()
                gathers[nb] = pltpu.async_copy(
                    table_hbm.at[idx_v.at[k + 1]], rows_v.at[nb], gsems[nb]
                )
            gathers[b].wait()
            wbs[b] = pltpu.async_copy(
                rows_v.at[b], out_hbm.at[pl.ds(base + k * CH, CH)], osems[b]
            )
        for w in wbs:
            if w is not None:
                w.wait()

    return gather_k


def _dot(a, b):
    return jnp.dot(a.astype(jnp.bfloat16), b.astype(jnp.bfloat16),
                   preferred_element_type=jnp.float32)


def _lstm_cell(iou, ct):
    i, o, u = iou[:, :H], iou[:, H:2 * H], iou[:, 2 * H:]
    c = jax.nn.sigmoid(i) * jnp.tanh(u) + ct
    h = jax.nn.sigmoid(o) * jnp.tanh(c)
    return h, c


def _levels(h, c, w_ref, b_ref, n_lev):
    for _ in range(n_lev):
        k = h.shape[0] // 2
        h2 = h.reshape(k, 2 * H)
        c2 = c.reshape(k, 2 * H)
        g = _dot(h2, w_ref[...]) + b_ref[...]
        f = jax.nn.sigmoid(g[:, :2 * H])
        fc = f * c2
        ct = fc[:, :H] + fc[:, H:]
        h, c = _lstm_cell(g[:, 2 * H:], ct)
    return h, c


def _blockA_body(x_ref, wl_ref, w_ref, b_ref, h_ref, c_ref):
    iou = _dot(x_ref[...], wl_ref[...]) + b_ref[...][:, 2 * H:]
    h, c = _lstm_cell(iou, 0.0)
    h, c = _levels(h, c, w_ref, b_ref, 4)
    h_ref[...] = h
    c_ref[...] = c


def _tailB_body(hin_ref, cin_ref, w_ref, b_ref, wo_ref, bo_ref, out_ref):
    h, c = _levels(hin_ref[...], cin_ref[...], w_ref, b_ref, 6)
    logits = jnp.dot(h, wo_ref[...], preferred_element_type=jnp.float32) + bo_ref[...]
    mx = jnp.max(logits, axis=-1, keepdims=True)
    e = jnp.exp(logits - mx)
    out_ref[...] = logits - mx - jnp.log(jnp.sum(e, axis=-1, keepdims=True))


def kernel(label, depth, batch, emb, W_iou, U_iou, b_iou, U_f_w, U_f_b, out_w, out_b):
    D = 10
    M = 2 ** (D + 1) - 1
    B = label.shape[0] // M
    NL = B * 2 ** D
    out_size = out_w.shape[0]

    leaf_labels = label.reshape(B, M)[:, M // 2:].reshape(-1).astype(jnp.int32)

    CH = 128
    info = plsc.get_sparse_core_info()
    NW = info.num_cores * info.num_subcores
    NG = 2
    NH = NL // NG
    gat = _make_sc_gather(emb.shape[0], NH, CH)
    xs = [gat(emb, leaf_labels[g * NH:(g + 1) * NH].reshape(NW, NH // (NW * CH), CH))
          for g in range(NG)]

    WlT = W_iou.T
    Wall = jnp.concatenate([U_f_w.T, U_iou.T], axis=1)
    ball = jnp.concatenate([U_f_b.reshape(1, 2 * H), b_iou.reshape(1, 3 * H)],
                           axis=1)

    T = 4
    LB = T * 2 ** D
    OB = T * 2 ** 6
    BH = B // NG
    hc = [pl.pallas_call(
        _blockA_body,
        grid=(BH // T,),
        in_specs=[
            pl.BlockSpec((LB, H), lambda i: (i, 0)),
            pl.BlockSpec((H, 3 * H), lambda i: (0, 0)),
            pl.BlockSpec((2 * H, 5 * H), lambda i: (0, 0)),
            pl.BlockSpec((1, 5 * H), lambda i: (0, 0)),
        ],
        out_specs=[
            pl.BlockSpec((OB, H), lambda i: (i, 0)),
            pl.BlockSpec((OB, H), lambda i: (i, 0)),
        ],
        out_shape=[jax.ShapeDtypeStruct((BH * 2 ** 6, H), jnp.float32)] * 2,
    )(xg, WlT, Wall, ball) for xg in xs]
    h6 = jnp.concatenate([p[0] for p in hc], axis=0)
    c6 = jnp.concatenate([p[1] for p in hc], axis=0)

    WoPad = jnp.zeros((H, 128), jnp.float32).at[:, :out_size].set(out_w.T)
    boPad = jnp.full((1, 128), -1e30, jnp.float32).at[0, :out_size].set(out_b)
    ls = pl.pallas_call(
        _tailB_body,
        grid=(1,),
        in_specs=[
            pl.BlockSpec((B * 2 ** 6, H), lambda i: (0, 0)),
            pl.BlockSpec((B * 2 ** 6, H), lambda i: (0, 0)),
            pl.BlockSpec((2 * H, 5 * H), lambda i: (0, 0)),
            pl.BlockSpec((1, 5 * H), lambda i: (0, 0)),
            pl.BlockSpec((H, 128), lambda i: (0, 0)),
            pl.BlockSpec((1, 128), lambda i: (0, 0)),
        ],
        out_specs=pl.BlockSpec((B, 128), lambda i: (0, 0)),
        out_shape=jax.ShapeDtypeStruct((B, 128), jnp.float32),
    )(h6, c6, Wall, ball, WoPad, boPad)
    return ls[:, :out_size]


# speedup vs baseline: 43.6964x; 1.0346x over previous
import functools

import jax
import jax.numpy as jnp
from jax import lax
from jax.experimental import pallas as pl
from jax.experimental.pallas import tpu as pltpu
from jax.experimental.pallas import tpu_sc as plsc

H = 256


@functools.lru_cache(maxsize=None)
def _make_sc_gather(V, N, CH):
    info = plsc.get_sparse_core_info()
    NC, NS = info.num_cores, info.num_subcores
    NW = NC * NS
    per_w = N // NW
    n_ch = per_w // CH
    assert per_w % CH == 0 and N % NW == 0 and CH <= 128
    mesh = plsc.VectorSubcoreMesh(core_axis_name="c", subcore_axis_name="s")

    @functools.partial(
        pl.kernel,
        mesh=mesh,
        out_type=jax.ShapeDtypeStruct((N, H), jnp.float32),
        scratch_types=[
            pltpu.VMEM((n_ch, CH), jnp.int32),
            pltpu.VMEM((2, CH, H), jnp.float32),
            pltpu.SemaphoreType.DMA,
            pltpu.SemaphoreType.DMA,
            pltpu.SemaphoreType.DMA,
            pltpu.SemaphoreType.DMA,
        ],
    )
    def gather_k(table_hbm, idx_hbm, out_hbm, idx_v, rows_v, g0, g1, o0, o1):
        wid = lax.axis_index("s") * NC + lax.axis_index("c")
        base = wid * per_w
        gsems = (g0, g1)
        osems = (o0, o1)

        pltpu.sync_copy(idx_hbm.at[wid], idx_v)
        gathers = [None, None]
        wbs = [None, None]
        gathers[0] = pltpu.async_copy(table_hbm.at[idx_v.at[0]], rows_v.at[0], g0)
        for k in range(n_ch):
            b = k % 2
            if k + 1 < n_ch:
                nb = 1 - b
                if wbs[nb] is not None:
                    wbs[nb].wait()
                gathers[nb] = pltpu.async_copy(
                    table_hbm.at[idx_v.at[k + 1]], rows_v.at[nb], gsems[nb]
                )
            gathers[b].wait()
            wbs[b] = pltpu.async_copy(
                rows_v.at[b], out_hbm.at[pl.ds(base + k * CH, CH)], osems[b]
            )
        for w in wbs:
            if w is not None:
                w.wait()

    return gather_k


def _dot(a, b):
    return jnp.dot(a.astype(jnp.bfloat16), b, preferred_element_type=jnp.float32)


def _lstm_cell(iou, ct):
    i, o, u = iou[:, :H], iou[:, H:2 * H], iou[:, 2 * H:]
    c = jax.nn.sigmoid(i) * jnp.tanh(u) + ct
    h = jax.nn.sigmoid(o) * jnp.tanh(c)
    return h, c


def _levels(h, c, w_ref, b_ref, n_lev):
    for _ in range(n_lev):
        k = h.shape[0] // 2
        h2 = h.reshape(k, 2 * H)
        c2 = c.reshape(k, 2 * H)
        g = _dot(h2, w_ref[...]) + b_ref[...]
        f = jax.nn.sigmoid(g[:, :2 * H])
        fc = f * c2
        ct = fc[:, :H] + fc[:, H:]
        h, c = _lstm_cell(g[:, 2 * H:], ct)
    return h, c


def _blockA_body(x_ref, wl_ref, w_ref, b_ref, h_ref, c_ref):
    iou = _dot(x_ref[...], wl_ref[...]) + b_ref[...][:, 2 * H:]
    h, c = _lstm_cell(iou, 0.0)
    h, c = _levels(h, c, w_ref, b_ref, 4)
    h_ref[...] = h
    c_ref[...] = c


def _tailB_body(h0_ref, c0_ref, h1_ref, c1_ref, w_ref, b_ref, wo_ref, bo_ref,
                out_ref):
    h = jnp.concatenate([h0_ref[...], h1_ref[...]], axis=0)
    c = jnp.concatenate([c0_ref[...], c1_ref[...]], axis=0)
    h, c = _levels(h, c, w_ref, b_ref, 6)
    logits = jnp.dot(h, wo_ref[...], preferred_element_type=jnp.float32) + bo_ref[...]
    mx = jnp.max(logits, axis=-1, keepdims=True)
    e = jnp.exp(logits - mx)
    out_ref[...] = logits - mx - jnp.log(jnp.sum(e, axis=-1, keepdims=True))


def kernel(label, depth, batch, emb, W_iou, U_iou, b_iou, U_f_w, U_f_b, out_w, out_b):
    D = 10
    M = 2 ** (D + 1) - 1
    B = label.shape[0] // M
    NL = B * 2 ** D
    out_size = out_w.shape[0]

    leaf_labels = label.reshape(B, M)[:, M // 2:].reshape(-1).astype(jnp.int32)

    CH = 128
    info = plsc.get_sparse_core_info()
    NW = info.num_cores * info.num_subcores
    NG = 2
    NH = NL // NG
    gat = _make_sc_gather(emb.shape[0], NH, CH)
    xs = [gat(emb, leaf_labels[g * NH:(g + 1) * NH].reshape(NW, NH // (NW * CH), CH))
          for g in range(NG)]

    WlT = W_iou.T.astype(jnp.bfloat16)
    Wall = jnp.concatenate([U_f_w, U_iou], axis=0).T.astype(jnp.bfloat16)
    ball = jnp.concatenate([U_f_b.reshape(1, 2 * H), b_iou.reshape(1, 3 * H)],
                           axis=1)

    T = 4
    LB = T * 2 ** D
    OB = T * 2 ** 6
    BH = B // NG
    hc = [pl.pallas_call(
        _blockA_body,
        grid=(BH // T,),
        in_specs=[
            pl.BlockSpec((LB, H), lambda i: (i, 0)),
            pl.BlockSpec((H, 3 * H), lambda i: (0, 0)),
            pl.BlockSpec((2 * H, 5 * H), lambda i: (0, 0)),
            pl.BlockSpec((1, 5 * H), lambda i: (0, 0)),
        ],
        out_specs=[
            pl.BlockSpec((OB, H), lambda i: (i, 0)),
            pl.BlockSpec((OB, H), lambda i: (i, 0)),
        ],
        out_shape=[jax.ShapeDtypeStruct((BH * 2 ** 6, H), jnp.float32)] * 2,
    )(xg, WlT, Wall, ball) for xg in xs]

    WoPad = jnp.zeros((H, 128), jnp.float32).at[:, :out_size].set(out_w.T)
    boPad = jnp.full((1, 128), -1e30, jnp.float32).at[0, :out_size].set(out_b)
    ls = pl.pallas_call(
        _tailB_body,
        grid=(1,),
        in_specs=[
            pl.BlockSpec((BH * 2 ** 6, H), lambda i: (0, 0)),
            pl.BlockSpec((BH * 2 ** 6, H), lambda i: (0, 0)),
            pl.BlockSpec((BH * 2 ** 6, H), lambda i: (0, 0)),
            pl.BlockSpec((BH * 2 ** 6, H), lambda i: (0, 0)),
            pl.BlockSpec((2 * H, 5 * H), lambda i: (0, 0)),
            pl.BlockSpec((1, 5 * H), lambda i: (0, 0)),
            pl.BlockSpec((H, 128), lambda i: (0, 0)),
            pl.BlockSpec((1, 128), lambda i: (0, 0)),
        ],
        out_specs=pl.BlockSpec((B, 128), lambda i: (0, 0)),
        out_shape=jax.ShapeDtypeStruct((B, 128), jnp.float32),
    )(hc[0][0], hc[0][1], hc[1][0], hc[1][1], Wall, ball, WoPad, boPad)
    return ls[:, :out_size]


# speedup vs baseline: 46.2547x; 1.0585x over previous
import functools

import jax
import jax.numpy as jnp
from jax import lax
from jax.experimental import pallas as pl
from jax.experimental.pallas import tpu as pltpu
from jax.experimental.pallas import tpu_sc as plsc

H = 256


@functools.lru_cache(maxsize=None)
def _make_sc_gather(V, N, CH):
    info = plsc.get_sparse_core_info()
    NC, NS = info.num_cores, info.num_subcores
    NW = NC * NS
    per_w = N // NW
    n_ch = per_w // CH
    assert per_w % CH == 0 and N % NW == 0 and CH <= 128
    mesh = plsc.VectorSubcoreMesh(core_axis_name="c", subcore_axis_name="s")

    @functools.partial(
        pl.kernel,
        mesh=mesh,
        out_type=jax.ShapeDtypeStruct((N, H), jnp.float32),
        scratch_types=[
            pltpu.VMEM((n_ch, CH), jnp.int32),
            pltpu.VMEM((2, CH, H), jnp.float32),
            pltpu.SemaphoreType.DMA,
            pltpu.SemaphoreType.DMA,
            pltpu.SemaphoreType.DMA,
            pltpu.SemaphoreType.DMA,
        ],
    )
    def gather_k(table_hbm, idx_hbm, out_hbm, idx_v, rows_v, g0, g1, o0, o1):
        wid = lax.axis_index("s") * NC + lax.axis_index("c")
        base = wid * per_w
        gsems = (g0, g1)
        osems = (o0, o1)

        pltpu.sync_copy(idx_hbm.at[wid], idx_v)
        gathers = [None, None]
        wbs = [None, None]
        gathers[0] = pltpu.async_copy(table_hbm.at[idx_v.at[0]], rows_v.at[0], g0)
        for k in range(n_ch):
            b = k % 2
            if k + 1 < n_ch:
                nb = 1 - b
                if wbs[nb] is not None:
                    wbs[nb].wait()
                gathers[nb] = pltpu.async_copy(
                    table_hbm.at[idx_v.at[k + 1]], rows_v.at[nb], gsems[nb]
                )
            gathers[b].wait()
            wbs[b] = pltpu.async_copy(
                rows_v.at[b], out_hbm.at[pl.ds(base + k * CH, CH)], osems[b]
            )
        for w in wbs:
            if w is not None:
                w.wait()

    return gather_k


def _dot(a, b):
    return jnp.dot(a.astype(jnp.bfloat16), b, preferred_element_type=jnp.float32)


def _sigmoid(x):
    return 0.5 * jnp.tanh(0.5 * x) + 0.5


def _lstm_cell(iou, ct):
    i, o, u = iou[:, :H], iou[:, H:2 * H], iou[:, 2 * H:]
    c = _sigmoid(i) * jnp.tanh(u) + ct
    h = _sigmoid(o) * jnp.tanh(c)
    return h, c


def _levels(h, c, w_ref, b_ref, n_lev):
    for _ in range(n_lev):
        k = h.shape[0] // 2
        h2 = h.reshape(k, 2 * H)
        c2 = c.reshape(k, 2 * H)
        g = _dot(h2, w_ref[...]) + b_ref[...]
        f = _sigmoid(g[:, :2 * H])
        fc = f * c2
        ct = fc[:, :H] + fc[:, H:]
        h, c = _lstm_cell(g[:, 2 * H:], ct)
    return h, c


def _blockA_body(x_ref, wl_ref, w_ref, b_ref, h_ref, c_ref):
    iou = _dot(x_ref[...], wl_ref[...]) + b_ref[...][:, 2 * H:]
    h, c = _lstm_cell(iou, 0.0)
    h, c = _levels(h, c, w_ref, b_ref, 4)
    h_ref[...] = h
    c_ref[...] = c


def _tailB_body(h0_ref, c0_ref, h1_ref, c1_ref, w_ref, b_ref, wo_ref, bo_ref,
                out_ref):
    h = jnp.concatenate([h0_ref[...], h1_ref[...]], axis=0)
    c = jnp.concatenate([c0_ref[...], c1_ref[...]], axis=0)
    h, c = _levels(h, c, w_ref, b_ref, 6)
    logits = jnp.dot(h, wo_ref[...], preferred_element_type=jnp.float32) + bo_ref[...]
    mx = jnp.max(logits, axis=-1, keepdims=True)
    e = jnp.exp(logits - mx)
    out_ref[...] = logits - mx - jnp.log(jnp.sum(e, axis=-1, keepdims=True))


def kernel(label, depth, batch, emb, W_iou, U_iou, b_iou, U_f_w, U_f_b, out_w, out_b):
    D = 10
    M = 2 ** (D + 1) - 1
    B = label.shape[0] // M
    NL = B * 2 ** D
    out_size = out_w.shape[0]

    leaf_labels = label.reshape(B, M)[:, M // 2:].reshape(-1).astype(jnp.int32)

    CH = 128
    info = plsc.get_sparse_core_info()
    NW = info.num_cores * info.num_subcores
    NG = 2
    NH = NL // NG
    gat = _make_sc_gather(emb.shape[0], NH, CH)
    xs = [gat(emb, leaf_labels[g * NH:(g + 1) * NH].reshape(NW, NH // (NW * CH), CH))
          for g in range(NG)]

    WlT = W_iou.T.astype(jnp.bfloat16)
    Wall = jnp.concatenate([U_f_w, U_iou], axis=0).T.astype(jnp.bfloat16)
    ball = jnp.concatenate([U_f_b.reshape(1, 2 * H), b_iou.reshape(1, 3 * H)],
                           axis=1)

    T = 4
    LB = T * 2 ** D
    OB = T * 2 ** 6
    BH = B // NG
    hc = [pl.pallas_call(
        _blockA_body,
        grid=(BH // T,),
        in_specs=[
            pl.BlockSpec((LB, H), lambda i: (i, 0)),
            pl.BlockSpec((H, 3 * H), lambda i: (0, 0)),
            pl.BlockSpec((2 * H, 5 * H), lambda i: (0, 0)),
            pl.BlockSpec((1, 5 * H), lambda i: (0, 0)),
        ],
        out_specs=[
            pl.BlockSpec((OB, H), lambda i: (i, 0)),
            pl.BlockSpec((OB, H), lambda i: (i, 0)),
        ],
        out_shape=[jax.ShapeDtypeStruct((BH * 2 ** 6, H), jnp.float32)] * 2,
    )(xg, WlT, Wall, ball) for xg in xs]

    WoPad = jnp.zeros((H, 128), jnp.float32).at[:, :out_size].set(out_w.T)
    boPad = jnp.full((1, 128), -1e30, jnp.float32).at[0, :out_size].set(out_b)
    ls = pl.pallas_call(
        _tailB_body,
        grid=(1,),
        in_specs=[
            pl.BlockSpec((BH * 2 ** 6, H), lambda i: (0, 0)),
            pl.BlockSpec((BH * 2 ** 6, H), lambda i: (0, 0)),
            pl.BlockSpec((BH * 2 ** 6, H), lambda i: (0, 0)),
            pl.BlockSpec((BH * 2 ** 6, H), lambda i: (0, 0)),
            pl.BlockSpec((2 * H, 5 * H), lambda i: (0, 0)),
            pl.BlockSpec((1, 5 * H), lambda i: (0, 0)),
            pl.BlockSpec((H, 128), lambda i: (0, 0)),
            pl.BlockSpec((1, 128), lambda i: (0, 0)),
        ],
        out_specs=pl.BlockSpec((B, 128), lambda i: (0, 0)),
        out_shape=jax.ShapeDtypeStruct((B, 128), jnp.float32),
    )(hc[0][0], hc[0][1], hc[1][0], hc[1][1], Wall, ball, WoPad, boPad)
    return ls[:, :out_size]


# speedup vs baseline: 46.9401x; 1.0148x over previous
import functools

import jax
import jax.numpy as jnp
from jax import lax
from jax.experimental import pallas as pl
from jax.experimental.pallas import tpu as pltpu
from jax.experimental.pallas import tpu_sc as plsc

H = 256


@functools.lru_cache(maxsize=None)
def _make_sc_gather(V, N, CH):
    info = plsc.get_sparse_core_info()
    NC, NS = info.num_cores, info.num_subcores
    NW = NC * NS
    per_w = N // NW
    n_ch = per_w // CH
    assert per_w % CH == 0 and N % NW == 0 and CH <= 128
    mesh = plsc.VectorSubcoreMesh(core_axis_name="c", subcore_axis_name="s")

    @functools.partial(
        pl.kernel,
        mesh=mesh,
        out_type=jax.ShapeDtypeStruct((N, H), jnp.float32),
        scratch_types=[
            pltpu.VMEM((n_ch, CH), jnp.int32),
            pltpu.VMEM((2, CH, H), jnp.float32),
            pltpu.SemaphoreType.DMA,
            pltpu.SemaphoreType.DMA,
            pltpu.SemaphoreType.DMA,
            pltpu.SemaphoreType.DMA,
        ],
    )
    def gather_k(table_hbm, idx_hbm, out_hbm, idx_v, rows_v, g0, g1, o0, o1):
        wid = lax.axis_index("s") * NC + lax.axis_index("c")
        base = wid * per_w
        gsems = (g0, g1)
        osems = (o0, o1)

        pltpu.sync_copy(idx_hbm.at[wid], idx_v)
        gathers = [None, None]
        wbs = [None, None]
        gathers[0] = pltpu.async_copy(table_hbm.at[idx_v.at[0]], rows_v.at[0], g0)
        for k in range(n_ch):
            b = k % 2
            if k + 1 < n_ch:
                nb = 1 - b
                if wbs[nb] is not None:
                    wbs[nb].wait()
                gathers[nb] = pltpu.async_copy(
                    table_hbm.at[idx_v.at[k + 1]], rows_v.at[nb], gsems[nb]
                )
            gathers[b].wait()
            wbs[b] = pltpu.async_copy(
                rows_v.at[b], out_hbm.at[pl.ds(base + k * CH, CH)], osems[b]
            )
        for w in wbs:
            if w is not None:
                w.wait()

    return gather_k


def _dot(a, b):
    return jnp.dot(a.astype(jnp.bfloat16), b, preferred_element_type=jnp.float32)


def _sigmoid(x):
    return 0.5 * jnp.tanh(0.5 * x) + 0.5


def _lstm_cell(iou, ct):
    i, o, u = iou[:, :H], iou[:, H:2 * H], iou[:, 2 * H:]
    c = _sigmoid(i) * jnp.tanh(u) + ct
    h = _sigmoid(o) * jnp.tanh(c)
    return h, c


def _levels(h, c, w_ref, b_ref, n_lev):
    for _ in range(n_lev):
        k = h.shape[0] // 2
        h2 = h.reshape(k, 2 * H)
        c2 = c.reshape(k, 2 * H)
        g = _dot(h2, w_ref[...]) + b_ref[...]
        f = _sigmoid(g[:, :2 * H])
        fc = f * c2
        ct = fc[:, :H] + fc[:, H:]
        h, c = _lstm_cell(g[:, 2 * H:], ct)
    return h, c


def _blockA_body(x_ref, wl_ref, w_ref, b_ref, h_ref, c_ref):
    iou = _dot(x_ref[...], wl_ref[...]) + b_ref[...][:, 2 * H:]
    h, c = _lstm_cell(iou, 0.0)
    h, c = _levels(h, c, w_ref, b_ref, 4)
    h_ref[...] = h
    c_ref[...] = c


def _blockA2_body(x_ref, wl_ref, w_ref, b_ref, h6a_ref, c6a_ref, wo_ref, bo_ref,
                  out_ref, h6s, c6s):
    i = pl.program_id(0)
    n = pl.num_programs(0)
    iou = _dot(x_ref[...], wl_ref[...]) + b_ref[...][:, 2 * H:]
    h, c = _lstm_cell(iou, 0.0)
    h, c = _levels(h, c, w_ref, b_ref, 4)
    ob = h.shape[0]
    h6s[pl.ds(i * ob, ob), :] = h
    c6s[pl.ds(i * ob, ob), :] = c

    @pl.when(i == n - 1)
    def _tail():
        hh = jnp.concatenate([h6a_ref[...], h6s[...]], axis=0)
        cc = jnp.concatenate([c6a_ref[...], c6s[...]], axis=0)
        hr, _ = _levels(hh, cc, w_ref, b_ref, 6)
        logits = (jnp.dot(hr, wo_ref[...], preferred_element_type=jnp.float32)
                  + bo_ref[...])
        mx = jnp.max(logits, axis=-1, keepdims=True)
        e = jnp.exp(logits - mx)
        out_ref[...] = logits - mx - jnp.log(jnp.sum(e, axis=-1, keepdims=True))


def kernel(label, depth, batch, emb, W_iou, U_iou, b_iou, U_f_w, U_f_b, out_w, out_b):
    D = 10
    M = 2 ** (D + 1) - 1
    B = label.shape[0] // M
    NL = B * 2 ** D
    out_size = out_w.shape[0]

    leaf_labels = label.reshape(B, M)[:, M // 2:].reshape(-1).astype(jnp.int32)

    CH = 128
    info = plsc.get_sparse_core_info()
    NW = info.num_cores * info.num_subcores
    NG = 2
    NH = NL // NG
    gat = _make_sc_gather(emb.shape[0], NH, CH)
    xs = [gat(emb, leaf_labels[g * NH:(g + 1) * NH].reshape(NW, NH // (NW * CH), CH))
          for g in range(NG)]

    WlT = W_iou.T.astype(jnp.bfloat16)
    Wall = jnp.concatenate([U_f_w, U_iou], axis=0).T.astype(jnp.bfloat16)
    ball = jnp.concatenate([U_f_b.reshape(1, 2 * H), b_iou.reshape(1, 3 * H)],
                           axis=1)

    T = 4
    LB = T * 2 ** D
    OB = T * 2 ** 6
    BH = B // NG
    N6 = BH * 2 ** 6
    h6a, c6a = pl.pallas_call(
        _blockA_body,
        grid=(BH // T,),
        in_specs=[
            pl.BlockSpec((LB, H), lambda i: (i, 0)),
            pl.BlockSpec((H, 3 * H), lambda i: (0, 0)),
            pl.BlockSpec((2 * H, 5 * H), lambda i: (0, 0)),
            pl.BlockSpec((1, 5 * H), lambda i: (0, 0)),
        ],
        out_specs=[
            pl.BlockSpec((OB, H), lambda i: (i, 0)),
            pl.BlockSpec((OB, H), lambda i: (i, 0)),
        ],
        out_shape=[jax.ShapeDtypeStruct((N6, H), jnp.float32)] * 2,
    )(xs[0], WlT, Wall, ball)

    WoPad = jnp.zeros((H, 128), jnp.float32).at[:, :out_size].set(out_w.T)
    boPad = jnp.full((1, 128), -1e30, jnp.float32).at[0, :out_size].set(out_b)
    ls = pl.pallas_call(
        _blockA2_body,
        grid=(BH // T,),
        in_specs=[
            pl.BlockSpec((LB, H), lambda i: (i, 0)),
            pl.BlockSpec((H, 3 * H), lambda i: (0, 0)),
            pl.BlockSpec((2 * H, 5 * H), lambda i: (0, 0)),
            pl.BlockSpec((1, 5 * H), lambda i: (0, 0)),
            pl.BlockSpec((N6, H), lambda i: (0, 0)),
            pl.BlockSpec((N6, H), lambda i: (0, 0)),
            pl.BlockSpec((H, 128), lambda i: (0, 0)),
            pl.BlockSpec((1, 128), lambda i: (0, 0)),
        ],
        out_specs=pl.BlockSpec((B, 128), lambda i: (0, 0)),
        out_shape=jax.ShapeDtypeStruct((B, 128), jnp.float32),
        scratch_shapes=[
            pltpu.VMEM((N6, H), jnp.float32),
            pltpu.VMEM((N6, H), jnp.float32),
        ],
    )(xs[1], WlT, Wall, ball, h6a, c6a, WoPad, boPad)
    return ls[:, :out_size]
